# baseline (speedup 1.0000x reference)
import functools

import jax
import jax.numpy as jnp
from jax import lax
from jax.experimental import pallas as pl
from jax.experimental.pallas import tpu as pltpu
from jax.experimental.pallas import tpu_sc as plsc

L = 16
N_ROWS = 128
N_COLS = 32768
NC = 2
NS = 16
SC_ROWS = NC * NS
TC_ROWS = N_ROWS - SC_ROWS

HALF = N_COLS // 2
HALF_CHUNKS = HALF // L
UNROLL = 8
NACC = 4

_INT_MAX = 0x7FFFFFFF


def _scan_half(buf, idx_base):
    def body(j, carry):
        acc = [list(acc_k) for acc_k in carry]
        for u in range(UNROLL):
            k = u % NACC
            a_k, i_k, v_k, c_k = acc[k]
            v = buf[pl.ds((j * UNROLL + u) * L, L)]
            a = jnp.abs(v)
            pred = a > a_k
            acc[k] = [
                jnp.where(pred, a, a_k),
                jnp.where(pred, c_k, i_k),
                jnp.where(pred, v, v_k),
                c_k + L * NACC,
            ]
        return tuple(tuple(acc_k) for acc_k in acc)

    init = tuple(
        (
            jnp.full((L,), -1.0, jnp.float32),
            jnp.zeros((L,), jnp.int32),
            jnp.zeros((L,), jnp.float32),
            lax.iota(jnp.int32, L) + jnp.int32(L * k + idx_base),
        )
        for k in range(NACC)
    )
    accs = lax.fori_loop(0, HALF_CHUNKS // UNROLL, body, init)

    best_a, best_i, best_v, _ = accs[0]
    for a_k, i_k, v_k, _ in accs[1:]:
        pred = (a_k > best_a) | ((a_k == best_a) & (i_k < best_i))
        best_a = jnp.where(pred, a_k, best_a)
        best_i = jnp.where(pred, i_k, best_i)
        best_v = jnp.where(pred, v_k, best_v)
    return best_a, best_i, best_v


def _merge(t0, t1):
    a0, i0, v0 = t0
    a1, i1, v1 = t1
    pred = (a1 > a0) | ((a1 == a0) & (i1 < i0))
    return (
        jnp.where(pred, a1, a0),
        jnp.where(pred, i1, i0),
        jnp.where(pred, v1, v0),
    )


def _sc_body(neuron_hbm, seg_hbm, out_hbm,
             buf0, buf1, nvec, contrib, red, outv, shared, sem0, sem1):
    cid = lax.axis_index("c")
    sid = lax.axis_index("s")
    row = cid * NS + sid

    c0 = pltpu.make_async_copy(seg_hbm.at[row, pl.ds(0, HALF)], buf0, sem0)
    c0.start()
    c1 = pltpu.make_async_copy(seg_hbm.at[row, pl.ds(HALF, HALF)], buf1, sem1)
    c1.start()

    pltpu.sync_copy(neuron_hbm.at[pl.ds(cid * NS, NS)], nvec)

    c0.wait()
    t0 = _scan_half(buf0, 0)
    c1.wait()
    t1 = _scan_half(buf1, HALF)
    best_a, best_i, best_v = _merge(t0, t1)

    lane = lax.iota(jnp.int32, L)
    for shift in (1, 2, 4, 8):
        perm = lane ^ shift
        other = (
            best_a.at[perm].get(mode="promise_in_bounds"),
            best_i.at[perm].get(mode="promise_in_bounds"),
            best_v.at[perm].get(mode="promise_in_bounds"),
        )
        best_a, best_i, best_v = _merge((best_a, best_i, best_v), other)

    p = 1.0 / (1.0 + jnp.exp(-best_v))
    contrib[...] = jnp.where(lane == sid, nvec[...] * p,
                             jnp.zeros((L,), jnp.float32))

    pltpu.sync_copy(contrib, shared.at[pl.ds(sid * L, L)])
    plsc.subcore_barrier()

    @pl.when(sid == 0)
    def _():
        pltpu.sync_copy(shared, red)
        acc = red[pl.ds(0, L)]
        for r in range(1, NS):
            acc = acc + red[pl.ds(r * L, L)]
        outv[...] = acc
        pltpu.sync_copy(outv, out_hbm.at[pl.ds(cid * NS, NS)])


_sc_call = functools.partial(
    pl.kernel,
    mesh=plsc.VectorSubcoreMesh(core_axis_name="c", subcore_axis_name="s"),
    out_type=jax.ShapeDtypeStruct((SC_ROWS,), jnp.float32),
    scratch_types=[
        pltpu.VMEM((HALF,), jnp.float32),
        pltpu.VMEM((HALF,), jnp.float32),
        pltpu.VMEM((L,), jnp.float32),
        pltpu.VMEM((L,), jnp.float32),
        pltpu.VMEM((NS * L,), jnp.float32),
        pltpu.VMEM((L,), jnp.float32),
        pltpu.VMEM_SHARED((NS * L,), jnp.float32),
        pltpu.SemaphoreType.DMA,
        pltpu.SemaphoreType.DMA,
    ],
)(_sc_body)


TC_BR = 32
TC_BC = 2048
TC_NBI = TC_ROWS // TC_BR
TC_NBJ = N_COLS // TC_BC
TC_OFF = SC_ROWS // TC_BR


def _tc_body(seg_ref, neuron_ref, out_ref, best_a, best_i, best_v):
    j = pl.program_id(1)

    x = seg_ref[...]
    a = jnp.abs(x)
    m = jnp.max(a, axis=1, keepdims=True)
    cols = lax.broadcasted_iota(jnp.int32, (TC_BR, TC_BC), 1) + j * TC_BC
    cand = jnp.where(a == m, cols, jnp.int32(_INT_MAX))
    ci = jnp.min(cand, axis=1, keepdims=True)
    v = jnp.sum(jnp.where(cols == ci, x, 0.0), axis=1, keepdims=True)

    @pl.when(j == 0)
    def _():
        best_a[...] = m
        best_i[...] = ci
        best_v[...] = v

    @pl.when(j > 0)
    def _():
        pa, pi, pv = best_a[...], best_i[...], best_v[...]
        pred = (m > pa) | ((m == pa) & (ci < pi))
        best_a[...] = jnp.where(pred, m, pa)
        best_i[...] = jnp.where(pred, ci, pi)
        best_v[...] = jnp.where(pred, v, pv)

    @pl.when(j == TC_NBJ - 1)
    def _():
        percent = 1.0 / (1.0 + jnp.exp(-best_v[...]))
        out_ref[...] = neuron_ref[...] * percent


_tc_call = pl.pallas_call(
    _tc_body,
    grid=(TC_NBI, TC_NBJ),
    in_specs=[
        pl.BlockSpec((TC_BR, TC_BC), lambda i, j: (i + TC_OFF, j)),
        pl.BlockSpec((TC_BR, 1), lambda i, j: (i + TC_OFF, 0)),
    ],
    out_specs=pl.BlockSpec((TC_BR, 1), lambda i, j: (i, 0)),
    out_shape=jax.ShapeDtypeStruct((TC_ROWS, 1), jnp.float32),
    scratch_shapes=[
        pltpu.VMEM((TC_BR, 1), jnp.float32),
        pltpu.VMEM((TC_BR, 1), jnp.int32),
        pltpu.VMEM((TC_BR, 1), jnp.float32),
    ],
    compiler_params=pltpu.CompilerParams(
        dimension_semantics=("arbitrary", "arbitrary")),
)


@jax.jit
def kernel(neuron_out, segment_out):
    sc_out = _sc_call(neuron_out, segment_out)
    tc_out = _tc_call(segment_out, neuron_out.reshape(N_ROWS, 1))
    return jnp.concatenate([sc_out, tc_out.reshape(TC_ROWS)])


# speedup vs baseline: 1.4699x; 1.4699x over previous
import functools

import jax
import jax.numpy as jnp
from jax import lax
from jax.experimental import pallas as pl
from jax.experimental.pallas import tpu as pltpu
from jax.experimental.pallas import tpu_sc as plsc

L = 16
N_ROWS = 128
N_COLS = 32768
NC = 2
NS = 16
SC_ROWS = NC * NS
TC_ROWS = N_ROWS - SC_ROWS

HALF = N_COLS // 2
HALF_CHUNKS = HALF // L
UNROLL = 8
NACC = 4

_INT_MAX = 0x7FFFFFFF


def _scan_half(buf, idx_base):
    def body(j, carry):
        acc = [list(acc_k) for acc_k in carry]
        for u in range(UNROLL):
            k = u % NACC
            a_k, i_k, v_k, c_k = acc[k]
            v = buf[pl.ds((j * UNROLL + u) * L, L)]
            a = jnp.abs(v)
            pred = a > a_k
            acc[k] = [
                jnp.where(pred, a, a_k),
                jnp.where(pred, c_k, i_k),
                jnp.where(pred, v, v_k),
                c_k + L * NACC,
            ]
        return tuple(tuple(acc_k) for acc_k in acc)

    init = tuple(
        (
            jnp.full((L,), -1.0, jnp.float32),
            jnp.zeros((L,), jnp.int32),
            jnp.zeros((L,), jnp.float32),
            lax.iota(jnp.int32, L) + jnp.int32(L * k + idx_base),
        )
        for k in range(NACC)
    )
    accs = lax.fori_loop(0, HALF_CHUNKS // UNROLL, body, init)

    best_a, best_i, best_v, _ = accs[0]
    for a_k, i_k, v_k, _ in accs[1:]:
        pred = (a_k > best_a) | ((a_k == best_a) & (i_k < best_i))
        best_a = jnp.where(pred, a_k, best_a)
        best_i = jnp.where(pred, i_k, best_i)
        best_v = jnp.where(pred, v_k, best_v)
    return best_a, best_i, best_v


def _merge(t0, t1):
    a0, i0, v0 = t0
    a1, i1, v1 = t1
    pred = (a1 > a0) | ((a1 == a0) & (i1 < i0))
    return (
        jnp.where(pred, a1, a0),
        jnp.where(pred, i1, i0),
        jnp.where(pred, v1, v0),
    )


def _sc_body(neuron_hbm, seg_hbm, out_hbm,
             buf0, buf1, nvec, contrib, red, outv, shared, sem0, sem1):
    cid = lax.axis_index("c")
    sid = lax.axis_index("s")
    row = cid * NS + sid

    c0 = pltpu.make_async_copy(seg_hbm.at[row, pl.ds(0, HALF)], buf0, sem0)
    c0.start()
    c1 = pltpu.make_async_copy(seg_hbm.at[row, pl.ds(HALF, HALF)], buf1, sem1)
    c1.start()

    pltpu.sync_copy(neuron_hbm.at[pl.ds(cid * NS, NS)], nvec)

    c0.wait()
    t0 = _scan_half(buf0, 0)
    c1.wait()
    t1 = _scan_half(buf1, HALF)
    best_a, best_i, best_v = _merge(t0, t1)

    lane = lax.iota(jnp.int32, L)
    for shift in (1, 2, 4, 8):
        perm = lane ^ shift
        other = (
            best_a.at[perm].get(mode="promise_in_bounds"),
            best_i.at[perm].get(mode="promise_in_bounds"),
            best_v.at[perm].get(mode="promise_in_bounds"),
        )
        best_a, best_i, best_v = _merge((best_a, best_i, best_v), other)

    p = 1.0 / (1.0 + jnp.exp(-best_v))
    contrib[...] = jnp.where(lane == sid, nvec[...] * p,
                             jnp.zeros((L,), jnp.float32))

    pltpu.sync_copy(contrib, shared.at[pl.ds(sid * L, L)])
    plsc.subcore_barrier()

    @pl.when(sid == 0)
    def _():
        pltpu.sync_copy(shared, red)
        acc = red[pl.ds(0, L)]
        for r in range(1, NS):
            acc = acc + red[pl.ds(r * L, L)]
        outv[...] = acc
        pltpu.sync_copy(outv, out_hbm.at[pl.ds(cid * NS, NS)])


_sc_call = functools.partial(
    pl.kernel,
    mesh=plsc.VectorSubcoreMesh(core_axis_name="c", subcore_axis_name="s"),
    out_type=jax.ShapeDtypeStruct((SC_ROWS,), jnp.float32),
    scratch_types=[
        pltpu.VMEM((HALF,), jnp.float32),
        pltpu.VMEM((HALF,), jnp.float32),
        pltpu.VMEM((L,), jnp.float32),
        pltpu.VMEM((L,), jnp.float32),
        pltpu.VMEM((NS * L,), jnp.float32),
        pltpu.VMEM((L,), jnp.float32),
        pltpu.VMEM_SHARED((NS * L,), jnp.float32),
        pltpu.SemaphoreType.DMA,
        pltpu.SemaphoreType.DMA,
    ],
)(_sc_body)


TC_BR = 32
TC_BC = 4096
TC_LANES = 128
TC_G = TC_BC // TC_LANES
TC_NBI = TC_ROWS // TC_BR
TC_NBJ = N_COLS // TC_BC
TC_OFF = SC_ROWS // TC_BR
TC_NACC = 4


def _tc_body(seg_ref, neuron_ref, out_ref, acc_a, acc_i, acc_v):
    j = pl.program_id(1)

    @pl.when(j == 0)
    def _():
        acc_a[...] = jnp.full((TC_NACC, TC_BR, TC_LANES), -1.0, jnp.float32)
        acc_i[...] = jnp.zeros((TC_NACC, TC_BR, TC_LANES), jnp.int32)
        acc_v[...] = jnp.zeros((TC_NACC, TC_BR, TC_LANES), jnp.float32)

    lane = lax.broadcasted_iota(jnp.int32, (TC_BR, TC_LANES), 1)
    col0 = j * TC_BC

    accs = [
        [acc_a[k], acc_i[k], acc_v[k]]
        for k in range(TC_NACC)
    ]
    for g in range(TC_G):
        k = g % TC_NACC
        a_k, i_k, v_k = accs[k]
        x = seg_ref[:, pl.ds(g * TC_LANES, TC_LANES)]
        a = jnp.abs(x)
        idx = lane + (col0 + g * TC_LANES)
        pred = a > a_k
        accs[k] = [
            jnp.where(pred, a, a_k),
            jnp.where(pred, idx, i_k),
            jnp.where(pred, x, v_k),
        ]
    for k in range(TC_NACC):
        acc_a[k], acc_i[k], acc_v[k] = accs[k]

    @pl.when(j == TC_NBJ - 1)
    def _():
        best_a, best_i, best_v = accs[0]
        for k in range(1, TC_NACC):
            a_k, i_k, v_k = accs[k]
            pred = (a_k > best_a) | ((a_k == best_a) & (i_k < best_i))
            best_a = jnp.where(pred, a_k, best_a)
            best_i = jnp.where(pred, i_k, best_i)
            best_v = jnp.where(pred, v_k, best_v)
        m = jnp.max(best_a, axis=1, keepdims=True)
        cand = jnp.where(best_a == m, best_i, jnp.int32(_INT_MAX))
        ci = jnp.min(cand, axis=1, keepdims=True)
        v = jnp.sum(jnp.where(best_i == ci, best_v, 0.0),
                    axis=1, keepdims=True)
        percent = 1.0 / (1.0 + jnp.exp(-v))
        out_ref[...] = neuron_ref[...] * percent


_tc_call = pl.pallas_call(
    _tc_body,
    grid=(TC_NBI, TC_NBJ),
    in_specs=[
        pl.BlockSpec((TC_BR, TC_BC), lambda i, j: (i + TC_OFF, j)),
        pl.BlockSpec((TC_BR, 1), lambda i, j: (i + TC_OFF, 0)),
    ],
    out_specs=pl.BlockSpec((TC_BR, 1), lambda i, j: (i, 0)),
    out_shape=jax.ShapeDtypeStruct((TC_ROWS, 1), jnp.float32),
    scratch_shapes=[
        pltpu.VMEM((TC_NACC, TC_BR, TC_LANES), jnp.float32),
        pltpu.VMEM((TC_NACC, TC_BR, TC_LANES), jnp.int32),
        pltpu.VMEM((TC_NACC, TC_BR, TC_LANES), jnp.float32),
    ],
    compiler_params=pltpu.CompilerParams(
        dimension_semantics=("arbitrary", "arbitrary")),
)


@jax.jit
def kernel(neuron_out, segment_out):
    sc_out = _sc_call(neuron_out, segment_out)
    tc_out = _tc_call(segment_out, neuron_out.reshape(N_ROWS, 1))
    tc_flat = tc_out.reshape(TC_ROWS)
    row_ids = lax.iota(jnp.int32, N_ROWS)
    sc_pad = jnp.pad(sc_out, (0, TC_ROWS))
    tc_pad = jnp.pad(tc_flat, (SC_ROWS, 0))
    return jnp.where(row_ids < SC_ROWS, sc_pad, tc_pad)


# speedup vs baseline: 1.7232x; 1.1723x over previous
import functools

import jax
import jax.numpy as jnp
from jax import lax
from jax.experimental import pallas as pl
from jax.experimental.pallas import tpu as pltpu
from jax.experimental.pallas import tpu_sc as plsc

L = 16
N_ROWS = 128
N_COLS = 32768
NC = 2
NS = 16
SC_ROWS = NC * NS
TC_ROWS = N_ROWS - SC_ROWS

HALF = N_COLS // 2
HALF_CHUNKS = HALF // L
UNROLL = 8
NACC = 4

_INT_MAX = 0x7FFFFFFF


def _scan_half(buf, idx_base):
    def body(j, carry):
        acc = [list(acc_k) for acc_k in carry]
        for u in range(UNROLL):
            k = u % NACC
            a_k, i_k, v_k, c_k = acc[k]
            v = buf[pl.ds((j * UNROLL + u) * L, L)]
            a = jnp.abs(v)
            pred = a > a_k
            acc[k] = [
                jnp.where(pred, a, a_k),
                jnp.where(pred, c_k, i_k),
                jnp.where(pred, v, v_k),
                c_k + L * NACC,
            ]
        return tuple(tuple(acc_k) for acc_k in acc)

    init = tuple(
        (
            jnp.full((L,), -1.0, jnp.float32),
            jnp.zeros((L,), jnp.int32),
            jnp.zeros((L,), jnp.float32),
            lax.iota(jnp.int32, L) + jnp.int32(L * k + idx_base),
        )
        for k in range(NACC)
    )
    accs = lax.fori_loop(0, HALF_CHUNKS // UNROLL, body, init)

    best_a, best_i, best_v, _ = accs[0]
    for a_k, i_k, v_k, _ in accs[1:]:
        pred = (a_k > best_a) | ((a_k == best_a) & (i_k < best_i))
        best_a = jnp.where(pred, a_k, best_a)
        best_i = jnp.where(pred, i_k, best_i)
        best_v = jnp.where(pred, v_k, best_v)
    return best_a, best_i, best_v


def _merge(t0, t1):
    a0, i0, v0 = t0
    a1, i1, v1 = t1
    pred = (a1 > a0) | ((a1 == a0) & (i1 < i0))
    return (
        jnp.where(pred, a1, a0),
        jnp.where(pred, i1, i0),
        jnp.where(pred, v1, v0),
    )


def _sc_body(neuron_hbm, seg_hbm, out_hbm,
             buf0, buf1, nvec, contrib, red, outv, shared, sem0, sem1):
    cid = lax.axis_index("c")
    sid = lax.axis_index("s")
    row = cid * NS + sid

    c0 = pltpu.make_async_copy(seg_hbm.at[row, pl.ds(0, HALF)], buf0, sem0)
    c0.start()
    c1 = pltpu.make_async_copy(seg_hbm.at[row, pl.ds(HALF, HALF)], buf1, sem1)
    c1.start()

    pltpu.sync_copy(neuron_hbm.at[pl.ds(cid * NS, NS)], nvec)

    c0.wait()
    t0 = _scan_half(buf0, 0)
    c1.wait()
    t1 = _scan_half(buf1, HALF)
    best_a, best_i, best_v = _merge(t0, t1)

    lane = lax.iota(jnp.int32, L)
    for shift in (1, 2, 4, 8):
        perm = lane ^ shift
        other = (
            best_a.at[perm].get(mode="promise_in_bounds"),
            best_i.at[perm].get(mode="promise_in_bounds"),
            best_v.at[perm].get(mode="promise_in_bounds"),
        )
        best_a, best_i, best_v = _merge((best_a, best_i, best_v), other)

    p = 1.0 / (1.0 + jnp.exp(-best_v))
    contrib[...] = jnp.where(lane == sid, nvec[...] * p,
                             jnp.zeros((L,), jnp.float32))

    pltpu.sync_copy(contrib, shared.at[pl.ds(sid * L, L)])
    plsc.subcore_barrier()

    @pl.when(sid == 0)
    def _():
        pltpu.sync_copy(shared, red)
        acc = red[pl.ds(0, L)]
        for r in range(1, NS):
            acc = acc + red[pl.ds(r * L, L)]
        outv[...] = acc
        pltpu.sync_copy(outv, out_hbm.at[pl.ds(cid * NS, NS)])


_sc_call = functools.partial(
    pl.kernel,
    mesh=plsc.VectorSubcoreMesh(core_axis_name="c", subcore_axis_name="s"),
    out_type=jax.ShapeDtypeStruct((N_ROWS,), jnp.float32),
    scratch_types=[
        pltpu.VMEM((HALF,), jnp.float32),
        pltpu.VMEM((HALF,), jnp.float32),
        pltpu.VMEM((L,), jnp.float32),
        pltpu.VMEM((L,), jnp.float32),
        pltpu.VMEM((NS * L,), jnp.float32),
        pltpu.VMEM((L,), jnp.float32),
        pltpu.VMEM_SHARED((NS * L,), jnp.float32),
        pltpu.SemaphoreType.DMA,
        pltpu.SemaphoreType.DMA,
    ],
)(_sc_body)


TC_BR = 32
TC_BC = 8192
TC_LANES = 128
TC_G = TC_BC // TC_LANES
TC_NBI = TC_ROWS // TC_BR
TC_NBJ = N_COLS // TC_BC
TC_OFF = SC_ROWS // TC_BR
TC_NACC = 8


def _tc_body(seg_ref, neuron_ref, out_ref, acc_a, acc_i, acc_v):
    j = pl.program_id(1)

    @pl.when(j == 0)
    def _():
        acc_a[...] = jnp.full((TC_NACC, TC_BR, TC_LANES), -1.0, jnp.float32)
        acc_i[...] = jnp.zeros((TC_NACC, TC_BR, TC_LANES), jnp.int32)
        acc_v[...] = jnp.zeros((TC_NACC, TC_BR, TC_LANES), jnp.float32)

    lane = lax.broadcasted_iota(jnp.int32, (TC_BR, TC_LANES), 1)
    col0 = j * TC_BC

    accs = [
        [acc_a[k], acc_i[k], acc_v[k]]
        for k in range(TC_NACC)
    ]
    for g in range(TC_G):
        k = g % TC_NACC
        a_k, i_k, v_k = accs[k]
        x = seg_ref[:, pl.ds(g * TC_LANES, TC_LANES)]
        a = jnp.abs(x)
        idx = lane + (col0 + g * TC_LANES)
        pred = a > a_k
        accs[k] = [
            jnp.where(pred, a, a_k),
            jnp.where(pred, idx, i_k),
            jnp.where(pred, x, v_k),
        ]
    for k in range(TC_NACC):
        acc_a[k], acc_i[k], acc_v[k] = accs[k]

    @pl.when(j == TC_NBJ - 1)
    def _():
        best_a, best_i, best_v = accs[0]
        for k in range(1, TC_NACC):
            a_k, i_k, v_k = accs[k]
            pred = (a_k > best_a) | ((a_k == best_a) & (i_k < best_i))
            best_a = jnp.where(pred, a_k, best_a)
            best_i = jnp.where(pred, i_k, best_i)
            best_v = jnp.where(pred, v_k, best_v)
        m = jnp.max(best_a, axis=1, keepdims=True)
        cand = jnp.where(best_a == m, best_i, jnp.int32(_INT_MAX))
        ci = jnp.min(cand, axis=1, keepdims=True)
        v = jnp.sum(jnp.where(best_i == ci, best_v, 0.0),
                    axis=1, keepdims=True)
        percent = 1.0 / (1.0 + jnp.exp(-v))
        out_ref[...] = neuron_ref[...] * percent


_tc_call = pl.pallas_call(
    _tc_body,
    grid=(TC_NBI, TC_NBJ),
    in_specs=[
        pl.BlockSpec((TC_BR, TC_BC), lambda i, j: (i + TC_OFF, j)),
        pl.BlockSpec((TC_BR, 1), lambda i, j: (i + TC_OFF, 0)),
    ],
    out_specs=pl.BlockSpec((TC_BR, 1), lambda i, j: (i + TC_OFF, 0)),
    out_shape=jax.ShapeDtypeStruct((N_ROWS, 1), jnp.float32),
    scratch_shapes=[
        pltpu.VMEM((TC_NACC, TC_BR, TC_LANES), jnp.float32),
        pltpu.VMEM((TC_NACC, TC_BR, TC_LANES), jnp.int32),
        pltpu.VMEM((TC_NACC, TC_BR, TC_LANES), jnp.float32),
    ],
    compiler_params=pltpu.CompilerParams(
        dimension_semantics=("arbitrary", "arbitrary")),
)


@jax.jit
def kernel(neuron_out, segment_out):
    sc_out = _sc_call(neuron_out, segment_out)
    tc_out = _tc_call(segment_out, neuron_out.reshape(N_ROWS, 1))
    row_ids = lax.iota(jnp.int32, N_ROWS)
    return jnp.where(row_ids < SC_ROWS, sc_out, tc_out.reshape(N_ROWS))


# speedup vs baseline: 1.8372x; 1.0662x over previous
import functools

import jax
import jax.numpy as jnp
from jax import lax
from jax.experimental import pallas as pl
from jax.experimental.pallas import tpu as pltpu
from jax.experimental.pallas import tpu_sc as plsc

L = 16
N_ROWS = 128
N_COLS = 32768
NC = 2
NS = 16
SC_ROWS = NC * NS
TC_ROWS = N_ROWS - SC_ROWS

HALF = N_COLS // 2
HALF_CHUNKS = HALF // L
UNROLL = 8
NACC = 4

_INT_MAX = 0x7FFFFFFF


def _scan_half(buf, idx_base):
    def body(j, carry):
        acc = [list(acc_k) for acc_k in carry]
        for u in range(UNROLL):
            k = u % NACC
            a_k, i_k, v_k, c_k = acc[k]
            v = buf[pl.ds((j * UNROLL + u) * L, L)]
            a = jnp.abs(v)
            pred = a > a_k
            acc[k] = [
                jnp.where(pred, a, a_k),
                jnp.where(pred, c_k, i_k),
                jnp.where(pred, v, v_k),
                c_k + L * NACC,
            ]
        return tuple(tuple(acc_k) for acc_k in acc)

    init = tuple(
        (
            jnp.full((L,), -1.0, jnp.float32),
            jnp.zeros((L,), jnp.int32),
            jnp.zeros((L,), jnp.float32),
            lax.iota(jnp.int32, L) + jnp.int32(L * k + idx_base),
        )
        for k in range(NACC)
    )
    accs = lax.fori_loop(0, HALF_CHUNKS // UNROLL, body, init)

    best_a, best_i, best_v, _ = accs[0]
    for a_k, i_k, v_k, _ in accs[1:]:
        pred = (a_k > best_a) | ((a_k == best_a) & (i_k < best_i))
        best_a = jnp.where(pred, a_k, best_a)
        best_i = jnp.where(pred, i_k, best_i)
        best_v = jnp.where(pred, v_k, best_v)
    return best_a, best_i, best_v


def _merge(t0, t1):
    a0, i0, v0 = t0
    a1, i1, v1 = t1
    pred = (a1 > a0) | ((a1 == a0) & (i1 < i0))
    return (
        jnp.where(pred, a1, a0),
        jnp.where(pred, i1, i0),
        jnp.where(pred, v1, v0),
    )


def _sc_body(neuron_hbm, seg_hbm, out_hbm,
             buf0, buf1, nvec, contrib, red, outv, shared, sem0, sem1):
    cid = lax.axis_index("c")
    sid = lax.axis_index("s")
    row = cid * NS + sid

    c0 = pltpu.make_async_copy(seg_hbm.at[row, pl.ds(0, HALF)], buf0, sem0)
    c0.start()
    c1 = pltpu.make_async_copy(seg_hbm.at[row, pl.ds(HALF, HALF)], buf1, sem1)
    c1.start()

    pltpu.sync_copy(neuron_hbm.at[pl.ds(cid * NS, NS)], nvec)

    c0.wait()
    t0 = _scan_half(buf0, 0)
    c1.wait()
    t1 = _scan_half(buf1, HALF)
    best_a, best_i, best_v = _merge(t0, t1)

    lane = lax.iota(jnp.int32, L)
    for shift in (1, 2, 4, 8):
        perm = lane ^ shift
        other = (
            best_a.at[perm].get(mode="promise_in_bounds"),
            best_i.at[perm].get(mode="promise_in_bounds"),
            best_v.at[perm].get(mode="promise_in_bounds"),
        )
        best_a, best_i, best_v = _merge((best_a, best_i, best_v), other)

    p = 1.0 / (1.0 + jnp.exp(-best_v))
    contrib[...] = jnp.where(lane == sid, nvec[...] * p,
                             jnp.zeros((L,), jnp.float32))

    pltpu.sync_copy(contrib, shared.at[pl.ds(sid * L, L)])
    plsc.subcore_barrier()

    @pl.when(sid == 0)
    def _():
        pltpu.sync_copy(shared, red)
        acc = red[pl.ds(0, L)]
        for r in range(1, NS):
            acc = acc + red[pl.ds(r * L, L)]
        outv[...] = acc
        pltpu.sync_copy(outv, out_hbm.at[pl.ds(cid * NS, NS)])


_sc_call = functools.partial(
    pl.kernel,
    mesh=plsc.VectorSubcoreMesh(core_axis_name="c", subcore_axis_name="s"),
    out_type=jax.ShapeDtypeStruct((N_ROWS,), jnp.float32),
    scratch_types=[
        pltpu.VMEM((HALF,), jnp.float32),
        pltpu.VMEM((HALF,), jnp.float32),
        pltpu.VMEM((L,), jnp.float32),
        pltpu.VMEM((L,), jnp.float32),
        pltpu.VMEM((NS * L,), jnp.float32),
        pltpu.VMEM((L,), jnp.float32),
        pltpu.VMEM_SHARED((NS * L,), jnp.float32),
        pltpu.SemaphoreType.DMA,
        pltpu.SemaphoreType.DMA,
    ],
)(_sc_body)


TC_BR = 32
TC_BC = 8192
TC_LANES = 128
TC_G = TC_BC // TC_LANES
TC_NBI = TC_ROWS // TC_BR
TC_NBJ = N_COLS // TC_BC
TC_OFF = SC_ROWS // TC_BR
TC_NACC = 8


def _tc_body(seg_ref, out_ref, acc_a, acc_i, acc_v, percent_s):
    i = pl.program_id(0)
    j = pl.program_id(1)

    @pl.when(j == 0)
    def _():
        acc_a[...] = jnp.full((TC_NACC, TC_BR, TC_LANES), -1.0, jnp.float32)
        acc_i[...] = jnp.zeros((TC_NACC, TC_BR, TC_LANES), jnp.int32)
        acc_v[...] = jnp.zeros((TC_NACC, TC_BR, TC_LANES), jnp.float32)

    lane = lax.broadcasted_iota(jnp.int32, (TC_BR, TC_LANES), 1)
    col0 = j * TC_BC

    accs = [
        [acc_a[k], acc_i[k], acc_v[k]]
        for k in range(TC_NACC)
    ]
    for g in range(TC_G):
        k = g % TC_NACC
        a_k, i_k, v_k = accs[k]
        x = seg_ref[:, pl.ds(g * TC_LANES, TC_LANES)]
        a = jnp.abs(x)
        idx = lane + (col0 + g * TC_LANES)
        pred = a > a_k
        accs[k] = [
            jnp.where(pred, a, a_k),
            jnp.where(pred, idx, i_k),
            jnp.where(pred, x, v_k),
        ]
    for k in range(TC_NACC):
        acc_a[k], acc_i[k], acc_v[k] = accs[k]

    @pl.when(j == TC_NBJ - 1)
    def _():
        best_a, best_i, best_v = accs[0]
        for k in range(1, TC_NACC):
            a_k, i_k, v_k = accs[k]
            pred = (a_k > best_a) | ((a_k == best_a) & (i_k < best_i))
            best_a = jnp.where(pred, a_k, best_a)
            best_i = jnp.where(pred, i_k, best_i)
            best_v = jnp.where(pred, v_k, best_v)
        m = jnp.max(best_a, axis=1, keepdims=True)
        cand = jnp.where(best_a == m, best_i, jnp.int32(_INT_MAX))
        ci = jnp.min(cand, axis=1, keepdims=True)
        v = jnp.sum(jnp.where(best_i == ci, best_v, 0.0),
                    axis=1, keepdims=True)
        percent = 1.0 / (1.0 + jnp.exp(-v))
        for ib in range(TC_NBI):
            @pl.when(i == ib)
            def _():
                percent_s[pl.ds((ib + TC_OFF) * TC_BR, TC_BR), :] = percent

    @pl.when((i == TC_NBI - 1) & (j == TC_NBJ - 1))
    def _():
        out_ref[...] = percent_s[...].T


_tc_call = pl.pallas_call(
    _tc_body,
    grid=(TC_NBI, TC_NBJ),
    in_specs=[
        pl.BlockSpec((TC_BR, TC_BC), lambda i, j: (i + TC_OFF, j)),
    ],
    out_specs=pl.BlockSpec((1, N_ROWS), lambda i, j: (0, 0)),
    out_shape=jax.ShapeDtypeStruct((1, N_ROWS), jnp.float32),
    scratch_shapes=[
        pltpu.VMEM((TC_NACC, TC_BR, TC_LANES), jnp.float32),
        pltpu.VMEM((TC_NACC, TC_BR, TC_LANES), jnp.int32),
        pltpu.VMEM((TC_NACC, TC_BR, TC_LANES), jnp.float32),
        pltpu.VMEM((N_ROWS, 1), jnp.float32),
    ],
    compiler_params=pltpu.CompilerParams(
        dimension_semantics=("arbitrary", "arbitrary")),
)


@jax.jit
def kernel(neuron_out, segment_out):
    sc_out = _sc_call(neuron_out, segment_out)
    tc_sig = _tc_call(segment_out)
    row_ids = lax.iota(jnp.int32, N_ROWS)
    return jnp.where(row_ids < SC_ROWS, sc_out,
                     tc_sig.reshape(N_ROWS) * neuron_out)
